# baseline (speedup 1.0000x reference)
import jax
import jax.numpy as jnp
from jax import lax
from jax.experimental import pallas as pl
from jax.experimental.pallas import tpu as pltpu
from jax.experimental.pallas import tpu_sc as plsc

D = 1024
S = 2048
B = 4
NW = 32
PPW = S // NW
C = 32
LANES = 16


def _emb_body(ids_hbm, wte_hbm, wpe_hbm, out_hbm, idx_v, wpe_v, rows_v, sem):
    wid = lax.axis_index("s") * 2 + lax.axis_index("c")
    base_pos = wid * PPW
    for pc in range(PPW // C):
        pos0 = base_pos + pc * C
        pltpu.sync_copy(wpe_hbm.at[pl.ds(pos0, C)], wpe_v)
        for b in range(B):
            tok0 = b * S + pos0
            pltpu.sync_copy(ids_hbm.at[pl.ds(tok0, C)], idx_v)
            pltpu.async_copy(wte_hbm.at[idx_v], rows_v, sem).wait()

            def add_row(r, carry):
                for j in range(D // LANES):
                    sl = (r, pl.ds(j * LANES, LANES))
                    rows_v[sl] = rows_v[sl] + wpe_v[sl]
                return carry

            lax.fori_loop(0, C, add_row, 0)
            pltpu.sync_copy(rows_v, out_hbm.at[pl.ds(tok0, C)])


def kernel(input_ids, wte, wpe):
    ids = input_ids.reshape(-1).astype(jnp.int32)
    mesh = plsc.VectorSubcoreMesh(core_axis_name="c", subcore_axis_name="s")
    f = pl.kernel(
        _emb_body,
        out_type=jax.ShapeDtypeStruct((B * S, D), jnp.float32),
        mesh=mesh,
        scratch_types=[
            pltpu.VMEM((C,), jnp.int32),
            pltpu.VMEM((C, D), jnp.float32),
            pltpu.VMEM((C, D), jnp.float32),
            pltpu.SemaphoreType.DMA,
        ],
    )
    out = f(ids, wte, wpe)
    return out.reshape(B, S, D)


# speedup vs baseline: 1.1054x; 1.1054x over previous
import jax
import jax.numpy as jnp
from jax import lax
from jax.experimental import pallas as pl
from jax.experimental.pallas import tpu as pltpu
from jax.experimental.pallas import tpu_sc as plsc

D = 1024
S = 2048
B = 4
NW = 32
PPW = S // NW
C = 16
NBUF = 3
LANES = 16


def _emb_body(ids_hbm, wte_hbm, wpe_hbm, out_hbm,
              idx_v, wpe_v, r0, r1, r2, g0, g1, g2, s0, s1, s2):
    rows = [r0, r1, r2]
    gsem = [g0, g1, g2]
    ssem = [s0, s1, s2]
    wid = lax.axis_index("s") * 2 + lax.axis_index("c")
    pos_base = wid * PPW

    pltpu.sync_copy(wpe_hbm.at[pl.ds(pos_base, PPW)], wpe_v)
    for b in range(B):
        pltpu.sync_copy(ids_hbm.at[pl.ds(b * S + pos_base, PPW)], idx_v.at[b])

    chunks = [(b, pc) for b in range(B) for pc in range(PPW // C)]
    n = len(chunks)
    hg = [None] * n
    hs = [None] * n

    def start_gather(j):
        b, pc = chunks[j]
        hg[j] = pltpu.async_copy(
            wte_hbm.at[idx_v.at[b, pl.ds(pc * C, C)]], rows[j % NBUF],
            gsem[j % NBUF])

    start_gather(0)
    for i in range(n):
        j = i + 1
        if j < n:
            if j >= NBUF:
                hs[j - NBUF].wait()
            start_gather(j)
        hg[i].wait()
        b, pc = chunks[i]
        buf = i % NBUF
        lp = pc * C

        def add_row(r, carry, _buf=buf, _lp=lp):
            for k in range(D // LANES):
                sl = pl.ds(k * LANES, LANES)
                rows[_buf][r, sl] = rows[_buf][r, sl] + wpe_v[_lp + r, sl]
            return carry

        lax.fori_loop(0, C, add_row, 0)
        tok0 = b * S + pos_base + pc * C
        hs[i] = pltpu.async_copy(rows[buf], out_hbm.at[pl.ds(tok0, C)],
                                 ssem[buf])
    for i in range(n - NBUF, n):
        hs[i].wait()


def kernel(input_ids, wte, wpe):
    ids = input_ids.reshape(-1).astype(jnp.int32)
    mesh = plsc.VectorSubcoreMesh(core_axis_name="c", subcore_axis_name="s")
    f = pl.kernel(
        _emb_body,
        out_type=jax.ShapeDtypeStruct((B * S, D), jnp.float32),
        mesh=mesh,
        scratch_types=[
            pltpu.VMEM((B, PPW), jnp.int32),
            pltpu.VMEM((PPW, D), jnp.float32),
            pltpu.VMEM((C, D), jnp.float32),
            pltpu.VMEM((C, D), jnp.float32),
            pltpu.VMEM((C, D), jnp.float32),
            pltpu.SemaphoreType.DMA,
            pltpu.SemaphoreType.DMA,
            pltpu.SemaphoreType.DMA,
            pltpu.SemaphoreType.DMA,
            pltpu.SemaphoreType.DMA,
            pltpu.SemaphoreType.DMA,
        ],
    )
    out = f(ids, wte, wpe)
    return out.reshape(B, S, D)


# speedup vs baseline: 1.2597x; 1.1396x over previous
import jax
import jax.numpy as jnp
from jax import lax
from jax.experimental import pallas as pl
from jax.experimental.pallas import tpu as pltpu
from jax.experimental.pallas import tpu_sc as plsc

D = 1024
S = 2048
B = 4
NW = 32
PPW = S // NW
C = 16
NPC = PPW // C
NBUF = 5
LANES = 16


def _emb_body(ids_hbm, wte_hbm, wpe_hbm, out_hbm,
              idx_v, w0, w1, r0, r1, r2, r3, r4,
              g0, g1, g2, g3, g4, s0, s1, s2, s3, s4, ws0, ws1):
    rows = [r0, r1, r2, r3, r4]
    gsem = [g0, g1, g2, g3, g4]
    ssem = [s0, s1, s2, s3, s4]
    wpe_c = [w0, w1]
    wsem = [ws0, ws1]
    wid = lax.axis_index("s") * 2 + lax.axis_index("c")
    pos_base = wid * PPW

    for b in range(B):
        pltpu.sync_copy(ids_hbm.at[pl.ds(b * S + pos_base, PPW)], idx_v.at[b])

    chunks = [(pc, b) for pc in range(NPC) for b in range(B)]
    n = len(chunks)
    hg = [None] * n
    hs = [None] * n
    hw = [None] * NPC

    def start_wpe(pc):
        hw[pc] = pltpu.async_copy(
            wpe_hbm.at[pl.ds(pos_base + pc * C, C)], wpe_c[pc % 2],
            wsem[pc % 2])

    def start_gather(j):
        pc, b = chunks[j]
        hg[j] = pltpu.async_copy(
            wte_hbm.at[idx_v.at[b, pl.ds(pc * C, C)]], rows[j % NBUF],
            gsem[j % NBUF])

    start_wpe(0)
    start_wpe(1)
    for j in range(NBUF - 1):
        start_gather(j)

    for i in range(n):
        j = i + NBUF - 1
        if j < n:
            hs[j - NBUF].wait() if j >= NBUF else None
            start_gather(j)
        pc, b = chunks[i]
        buf = i % NBUF
        if b == 0:
            hw[pc].wait()
        hg[i].wait()
        wbuf = pc % 2

        def add_row(r, carry, _buf=buf, _wbuf=wbuf):
            for k in range(D // LANES):
                sl = pl.ds(k * LANES, LANES)
                rows[_buf][r, sl] = rows[_buf][r, sl] + wpe_c[_wbuf][r, sl]
            return carry

        lax.fori_loop(0, C, add_row, 0)
        if b == B - 1 and pc + 2 < NPC:
            start_wpe(pc + 2)
        tok0 = b * S + pos_base + pc * C
        hs[i] = pltpu.async_copy(rows[buf], out_hbm.at[pl.ds(tok0, C)],
                                 ssem[buf])
    for i in range(n - NBUF, n):
        hs[i].wait()


def kernel(input_ids, wte, wpe):
    ids = input_ids.reshape(-1).astype(jnp.int32)
    mesh = plsc.VectorSubcoreMesh(core_axis_name="c", subcore_axis_name="s")
    f = pl.kernel(
        _emb_body,
        out_type=jax.ShapeDtypeStruct((B * S, D), jnp.float32),
        mesh=mesh,
        scratch_types=(
            [pltpu.VMEM((B, PPW), jnp.int32)]
            + [pltpu.VMEM((C, D), jnp.float32) for _ in range(2)]
            + [pltpu.VMEM((C, D), jnp.float32) for _ in range(NBUF)]
            + [pltpu.SemaphoreType.DMA for _ in range(2 * NBUF + 2)]
        ),
    )
    out = f(ids, wte, wpe)
    return out.reshape(B, S, D)


# speedup vs baseline: 1.3537x; 1.0746x over previous
import jax
import jax.numpy as jnp
from jax import lax
from jax.experimental import pallas as pl
from jax.experimental.pallas import tpu as pltpu
from jax.experimental.pallas import tpu_sc as plsc

D = 1024
S = 2048
B = 4
NW = 32
PPW = S // NW
C = 16
NPC = PPW // C
NBUF = 5
LANES = 16


def _emb_body(ids_hbm, wte_hbm, wpe_hbm, out_hbm,
              idx_v, w0, w1, r0, r1, r2, r3, r4,
              g0, g1, g2, g3, g4, s0, s1, s2, s3, s4, ws0, ws1):
    rows = [r0, r1, r2, r3, r4]
    gsem = [g0, g1, g2, g3, g4]
    ssem = [s0, s1, s2, s3, s4]
    wpe_c = [w0, w1]
    wsem = [ws0, ws1]
    wid = lax.axis_index("s") * 2 + lax.axis_index("c")
    pos_base = wid * PPW

    for b in range(B):
        pltpu.sync_copy(ids_hbm.at[pl.ds(b * S + pos_base, PPW)], idx_v.at[b])

    chunks = [(pc, b) for pc in range(NPC) for b in range(B)]
    n = len(chunks)
    hg = [None] * n
    hs = [None] * n
    hw = [None] * NPC

    def start_wpe(pc):
        hw[pc] = pltpu.async_copy(
            wpe_hbm.at[pl.ds(pos_base + pc * C, C)], wpe_c[pc % 2],
            wsem[pc % 2])

    def start_gather(j):
        pc, b = chunks[j]
        hg[j] = pltpu.async_copy(
            wte_hbm.at[idx_v.at[b, pl.ds(pc * C, C)]], rows[j % NBUF],
            gsem[j % NBUF])

    start_wpe(0)
    start_wpe(1)
    for j in range(NBUF - 1):
        start_gather(j)

    for i in range(n):
        j = i + NBUF - 1
        if j < n:
            hs[j - NBUF].wait() if j >= NBUF else None
            start_gather(j)
        pc, b = chunks[i]
        buf = i % NBUF
        if b == 0:
            hw[pc].wait()
        hg[i].wait()
        wbuf = pc % 2

        def add_row(r, carry, _buf=buf, _wbuf=wbuf):
            for k in range(D // LANES):
                sl = pl.ds(k * LANES, LANES)
                plsc.addupdate(rows[_buf].at[r, sl], wpe_c[_wbuf][r, sl])
            return carry

        lax.fori_loop(0, C, add_row, 0)
        if b == B - 1 and pc + 2 < NPC:
            start_wpe(pc + 2)
        tok0 = b * S + pos_base + pc * C
        hs[i] = pltpu.async_copy(rows[buf], out_hbm.at[pl.ds(tok0, C)],
                                 ssem[buf])
    for i in range(n - NBUF, n):
        hs[i].wait()


def kernel(input_ids, wte, wpe):
    ids = input_ids.reshape(-1).astype(jnp.int32)
    mesh = plsc.VectorSubcoreMesh(core_axis_name="c", subcore_axis_name="s")
    f = pl.kernel(
        _emb_body,
        out_type=jax.ShapeDtypeStruct((B * S, D), jnp.float32),
        mesh=mesh,
        scratch_types=(
            [pltpu.VMEM((B, PPW), jnp.int32)]
            + [pltpu.VMEM((C, D), jnp.float32) for _ in range(2)]
            + [pltpu.VMEM((C, D), jnp.float32) for _ in range(NBUF)]
            + [pltpu.SemaphoreType.DMA for _ in range(2 * NBUF + 2)]
        ),
    )
    out = f(ids, wte, wpe)
    return out.reshape(B, S, D)


# speedup vs baseline: 1.3651x; 1.0084x over previous
import jax
import jax.numpy as jnp
from jax import lax
from jax.experimental import pallas as pl
from jax.experimental.pallas import tpu as pltpu
from jax.experimental.pallas import tpu_sc as plsc

D = 1024
S = 2048
B = 4
NW = 32
PPW = S // NW
C = 8
NPC = PPW // C
NGRP = 3
LANES = 16


def _emb_body(ids_hbm, wte_hbm, wpe_hbm, out_hbm, idx_v, *scratch):
    rows = scratch[:NGRP * B]
    wpe_c = scratch[NGRP * B:NGRP * B + 2]
    gsem = scratch[NGRP * B + 2:2 * NGRP * B + 2]
    ssem = scratch[2 * NGRP * B + 2:3 * NGRP * B + 2]
    wsem = scratch[3 * NGRP * B + 2:]
    wid = lax.axis_index("s") * 2 + lax.axis_index("c")
    pos_base = wid * PPW

    for b in range(B):
        pltpu.sync_copy(ids_hbm.at[pl.ds(b * S + pos_base, PPW)], idx_v.at[b])

    hg = [[None] * B for _ in range(NPC)]
    hs = [[None] * B for _ in range(NPC)]
    hw = [None] * NPC

    def start_wpe(g):
        hw[g] = pltpu.async_copy(
            wpe_hbm.at[pl.ds(pos_base + g * C, C)], wpe_c[g % 2],
            wsem[g % 2])

    def start_gather(g, b):
        k = (g % NGRP) * B + b
        hg[g][b] = pltpu.async_copy(
            wte_hbm.at[idx_v.at[b, pl.ds(g * C, C)]], rows[k], gsem[k])

    start_wpe(0)
    start_wpe(1)
    for g in range(2):
        for b in range(B):
            start_gather(g, b)

    for g in range(NPC):
        bufs = [(g % NGRP) * B + b for b in range(B)]
        wb = g % 2
        hw[g].wait()
        for b in range(B):
            hg[g][b].wait()

        def add_row(r, carry, _bufs=bufs, _wb=wb):
            for k in range(D // LANES):
                sl = pl.ds(k * LANES, LANES)
                v = wpe_c[_wb][r, sl]
                for j in _bufs:
                    plsc.addupdate(rows[j].at[r, sl], v)
            return carry

        lax.fori_loop(0, C, add_row, 0)

        for b in range(B):
            tok0 = b * S + pos_base + g * C
            hs[g][b] = pltpu.async_copy(
                rows[bufs[b]], out_hbm.at[pl.ds(tok0, C)], ssem[bufs[b]])

        if g + 2 < NPC:
            start_wpe(g + 2)
            for b in range(B):
                if g >= 1:
                    hs[g - 1][b].wait()
                start_gather(g + 2, b)

    for g in range(NPC - 3, NPC):
        for b in range(B):
            hs[g][b].wait()


def kernel(input_ids, wte, wpe):
    ids = input_ids.reshape(-1).astype(jnp.int32)
    mesh = plsc.VectorSubcoreMesh(core_axis_name="c", subcore_axis_name="s")
    f = pl.kernel(
        _emb_body,
        out_type=jax.ShapeDtypeStruct((B * S, D), jnp.float32),
        mesh=mesh,
        scratch_types=(
            [pltpu.VMEM((B, PPW), jnp.int32)]
            + [pltpu.VMEM((C, D), jnp.float32) for _ in range(NGRP * B)]
            + [pltpu.VMEM((C, D), jnp.float32) for _ in range(2)]
            + [pltpu.SemaphoreType.DMA for _ in range(2 * NGRP * B + 2)]
        ),
    )
    out = f(ids, wte, wpe)
    return out.reshape(B, S, D)


# speedup vs baseline: 1.4837x; 1.0869x over previous
import jax
import jax.numpy as jnp
from jax import lax
from jax.experimental import pallas as pl
from jax.experimental.pallas import tpu as pltpu
from jax.experimental.pallas import tpu_sc as plsc

D = 1024
S = 2048
B = 4
NW = 32
PPW = S // NW
C = 8
NPC = PPW // C
NGRP = 3
LANES = 16


def _emb_body(ids_hbm, wte_hbm, wpe_hbm, out_hbm, idx_v, *scratch):
    rows = scratch[:NGRP * B]
    wpe_c = scratch[NGRP * B:NGRP * B + 2]
    gsem = scratch[NGRP * B + 2:2 * NGRP * B + 2]
    ssem = scratch[2 * NGRP * B + 2:3 * NGRP * B + 2]
    wsem = scratch[3 * NGRP * B + 2:]
    wid = lax.axis_index("s") * 2 + lax.axis_index("c")
    pos_base = wid * PPW

    for b in range(B):
        pltpu.sync_copy(ids_hbm.at[pl.ds(b * S + pos_base, PPW)], idx_v.at[b])

    hg = [[None] * B for _ in range(NPC)]
    hs = [[None] * B for _ in range(NPC)]
    hw = [None] * NPC

    def start_wpe(g):
        hw[g] = pltpu.async_copy(
            wpe_hbm.at[pl.ds(pos_base + g * C, C)], wpe_c[g % 2],
            wsem[g % 2])

    def start_gather(g, b):
        k = (g % NGRP) * B + b
        hg[g][b] = pltpu.async_copy(
            wte_hbm.at[idx_v.at[b, pl.ds(g * C, C)]], rows[k], gsem[k])

    start_wpe(0)
    start_wpe(1)
    for g in range(2):
        for b in range(B):
            start_gather(g, b)

    for g in range(NPC):
        bufs = [(g % NGRP) * B + b for b in range(B)]
        wb = g % 2
        hw[g].wait()
        for b in range(B):
            hg[g][b].wait()

        def add_row(r, carry, _bufs=bufs, _wb=wb):
            def add_kb(kb, carry2):
                for k in range(8):
                    sl = pl.ds(kb * (8 * LANES) + k * LANES, LANES)
                    v = wpe_c[_wb][r, sl]
                    for j in _bufs:
                        plsc.addupdate(rows[j].at[r, sl], v)
                return carry2

            return lax.fori_loop(0, D // (8 * LANES), add_kb, carry)

        lax.fori_loop(0, C, add_row, 0)

        for b in range(B):
            tok0 = b * S + pos_base + g * C
            hs[g][b] = pltpu.async_copy(
                rows[bufs[b]], out_hbm.at[pl.ds(tok0, C)], ssem[bufs[b]])

        if g + 2 < NPC:
            start_wpe(g + 2)
            for b in range(B):
                if g >= 1:
                    hs[g - 1][b].wait()
                start_gather(g + 2, b)

    for g in range(NPC - 3, NPC):
        for b in range(B):
            hs[g][b].wait()


def kernel(input_ids, wte, wpe):
    ids = input_ids.reshape(-1).astype(jnp.int32)
    mesh = plsc.VectorSubcoreMesh(core_axis_name="c", subcore_axis_name="s")
    f = pl.kernel(
        _emb_body,
        out_type=jax.ShapeDtypeStruct((B * S, D), jnp.float32),
        mesh=mesh,
        scratch_types=(
            [pltpu.VMEM((B, PPW), jnp.int32)]
            + [pltpu.VMEM((C, D), jnp.float32) for _ in range(NGRP * B)]
            + [pltpu.VMEM((C, D), jnp.float32) for _ in range(2)]
            + [pltpu.SemaphoreType.DMA for _ in range(2 * NGRP * B + 2)]
        ),
    )
    out = f(ids, wte, wpe)
    return out.reshape(B, S, D)


# speedup vs baseline: 1.5836x; 1.0673x over previous
import jax
import jax.numpy as jnp
from jax import lax
from jax.experimental import pallas as pl
from jax.experimental.pallas import tpu as pltpu
from jax.experimental.pallas import tpu_sc as plsc

D = 1024
S = 2048
B = 4
NW = 32
PPW = S // NW
C = 8
NPC = PPW // C
NGRP = 3
LANES = 16
KB = 8


def _emb_body(ids_hbm, wte_hbm, wpe_hbm, out_hbm,
              idx_v, rowsb, wpeb, gsem, ssem, wsem):
    wid = lax.axis_index("s") * 2 + lax.axis_index("c")
    pos_base = wid * PPW

    for b in range(B):
        pltpu.sync_copy(ids_hbm.at[pl.ds(b * S + pos_base, PPW)], idx_v.at[b])

    def wpe_copy(g):
        return pltpu.make_async_copy(
            wpe_hbm.at[pl.ds(pos_base + g * C, C)], wpeb.at[g % 2],
            wsem.at[g % 2])

    def gather_copy(g, b):
        return pltpu.make_async_copy(
            wte_hbm.at[idx_v.at[b, pl.ds(g * C, C)]], rowsb.at[g % NGRP, b],
            gsem.at[g % NGRP, b])

    def store_copy(g, b):
        return pltpu.make_async_copy(
            rowsb.at[g % NGRP, b],
            out_hbm.at[pl.ds(b * S + pos_base + g * C, C)],
            ssem.at[g % NGRP, b])

    wpe_copy(0).start()
    wpe_copy(1).start()
    for g in range(2):
        for b in range(B):
            gather_copy(g, b).start()

    def group_body(g, carry):
        p = g % NGRP
        wb = g % 2
        wpe_copy(g).wait()

        def wait_g(b, c):
            gather_copy(g, b).wait()
            return c

        lax.fori_loop(0, B, wait_g, 0)

        def add_row(r, c1):
            def add_kb(kb, c2):
                for k in range(KB):
                    sl = pl.ds(kb * (KB * LANES) + k * LANES, LANES)
                    v = wpeb[wb, r, sl]
                    for b in range(B):
                        plsc.addupdate(rowsb.at[p, b, r, sl], v)
                return c2

            return lax.fori_loop(0, D // (KB * LANES), add_kb, c1)

        lax.fori_loop(0, C, add_row, 0)

        def issue_store(b, c):
            store_copy(g, b).start()
            return c

        lax.fori_loop(0, B, issue_store, 0)

        def prefetch(_):
            wpe_copy(g + 2).start()

            def next_gather(b, c):
                def drain(_):
                    store_copy(g - 1, b).wait()
                    return 0

                lax.cond(g >= 1, drain, lambda _: 0, 0)
                gather_copy(g + 2, b).start()
                return c

            lax.fori_loop(0, B, next_gather, 0)
            return 0

        lax.cond(g + 2 < NPC, prefetch, lambda _: 0, 0)
        return carry

    lax.fori_loop(0, NPC, group_body, 0)

    def drain_tail(g, c):
        def drain_b(b, c2):
            store_copy(g, b).wait()
            return c2

        lax.fori_loop(0, B, drain_b, 0)
        return c

    lax.fori_loop(NPC - NGRP, NPC, drain_tail, 0)


def kernel(input_ids, wte, wpe):
    ids = input_ids.reshape(-1).astype(jnp.int32)
    mesh = plsc.VectorSubcoreMesh(core_axis_name="c", subcore_axis_name="s")
    f = pl.kernel(
        _emb_body,
        out_type=jax.ShapeDtypeStruct((B * S, D), jnp.float32),
        mesh=mesh,
        scratch_types=(
            pltpu.VMEM((B, PPW), jnp.int32),
            pltpu.VMEM((NGRP, B, C, D), jnp.float32),
            pltpu.VMEM((2, C, D), jnp.float32),
            pltpu.SemaphoreType.DMA((NGRP, B)),
            pltpu.SemaphoreType.DMA((NGRP, B)),
            pltpu.SemaphoreType.DMA((2,)),
        ),
    )
    out = f(ids, wte, wpe)
    return out.reshape(B, S, D)


# speedup vs baseline: 1.5960x; 1.0079x over previous
import jax
import jax.numpy as jnp
from jax import lax
from jax.experimental import pallas as pl
from jax.experimental.pallas import tpu as pltpu
from jax.experimental.pallas import tpu_sc as plsc

D = 1024
S = 2048
B = 4
NW = 32
PPW = S // NW
C = 8
NPC = PPW // C
NGRP = 3
LANES = 16
KB = 8


def _emb_body(ids_hbm, wte_hbm, wpe_hbm, out_hbm,
              idx_v, rowsb, wpeb, gsem, ssem, wsem):
    wid = lax.axis_index("s") * 2 + lax.axis_index("c")
    pos_base = wid * PPW

    for b in range(B):
        pltpu.sync_copy(ids_hbm.at[b, pl.ds(pos_base, PPW)], idx_v.at[b])

    def wpe_copy(g):
        return pltpu.make_async_copy(
            wpe_hbm.at[pl.ds(pos_base + g * C, C)], wpeb.at[g % 2],
            wsem.at[g % 2])

    def gather_copy(g, b):
        return pltpu.make_async_copy(
            wte_hbm.at[idx_v.at[b, pl.ds(g * C, C)]], rowsb.at[g % NGRP, b],
            gsem.at[g % NGRP, b])

    def store_copy(g, b):
        return pltpu.make_async_copy(
            rowsb.at[g % NGRP, b],
            out_hbm.at[pl.ds(b * S + pos_base + g * C, C)],
            ssem.at[g % NGRP, b])

    wpe_copy(0).start()
    wpe_copy(1).start()
    for g in range(2):
        for b in range(B):
            gather_copy(g, b).start()

    def group_body(g, carry):
        p = g % NGRP
        wb = g % 2
        wpe_copy(g).wait()

        def wait_g(b, c):
            gather_copy(g, b).wait()
            return c

        lax.fori_loop(0, B, wait_g, 0)

        def add_row(r, c1):
            for k in range(D // LANES):
                sl = pl.ds(k * LANES, LANES)
                v = wpeb[wb, r, sl]
                for b in range(B):
                    plsc.addupdate(rowsb.at[p, b, r, sl], v)
            return c1

        lax.fori_loop(0, C, add_row, 0)

        def issue_store(b, c):
            store_copy(g, b).start()
            return c

        lax.fori_loop(0, B, issue_store, 0)

        def prefetch(_):
            wpe_copy(g + 2).start()

            def next_gather(b, c):
                def drain(_):
                    store_copy(g - 1, b).wait()
                    return 0

                lax.cond(g >= 1, drain, lambda _: 0, 0)
                gather_copy(g + 2, b).start()
                return c

            lax.fori_loop(0, B, next_gather, 0)
            return 0

        lax.cond(g + 2 < NPC, prefetch, lambda _: 0, 0)
        return carry

    lax.fori_loop(0, NPC, group_body, 0)

    def drain_tail(g, c):
        def drain_b(b, c2):
            store_copy(g, b).wait()
            return c2

        lax.fori_loop(0, B, drain_b, 0)
        return c

    lax.fori_loop(NPC - NGRP, NPC, drain_tail, 0)


def kernel(input_ids, wte, wpe):
    ids = input_ids.astype(jnp.int32)
    mesh = plsc.VectorSubcoreMesh(core_axis_name="c", subcore_axis_name="s")
    f = pl.kernel(
        _emb_body,
        out_type=jax.ShapeDtypeStruct((B * S, D), jnp.float32),
        mesh=mesh,
        scratch_types=(
            pltpu.VMEM((B, PPW), jnp.int32),
            pltpu.VMEM((NGRP, B, C, D), jnp.float32),
            pltpu.VMEM((2, C, D), jnp.float32),
            pltpu.SemaphoreType.DMA((NGRP, B)),
            pltpu.SemaphoreType.DMA((NGRP, B)),
            pltpu.SemaphoreType.DMA((2,)),
        ),
    )
    out = f(ids, wte, wpe)
    return out.reshape(B, S, D)
